# baseline (speedup 1.0000x reference)
import functools

import jax
import jax.numpy as jnp
from jax import lax
from jax.experimental import pallas as pl
from jax.experimental.pallas import tpu as pltpu
from jax.experimental.pallas import tpu_sc as plsc

NUM_FIELDS = 26
VOCAB = 100000
DIM = 32
BATCH = 16384

NC = 2
NS = 16
NW = NC * NS
PAIRS_PER_W = (NUM_FIELDS * DIM) // NW
QOUT = BATCH // 4
G16 = QOUT // 16
NCH = 4
VCH = 25088
VTAIL = (VOCAB // 128) * 128
VCHS = [(0, VCH), (VCH, VCH), (2 * VCH, VCH), (3 * VCH, VTAIL - 3 * VCH)]
TAIL = VOCAB - VTAIL

_mesh = plsc.VectorSubcoreMesh(core_axis_name="c", subcore_axis_name="s")


@functools.partial(
    pl.kernel,
    out_type=jax.ShapeDtypeStruct((NUM_FIELDS, DIM, BATCH), jnp.float32),
    mesh=_mesh,
    scratch_types=[
        pltpu.VMEM((BATCH,), jnp.int32),
        pltpu.VMEM((VOCAB,), jnp.float32),
        pltpu.VMEM((2, QOUT), jnp.float32),
        pltpu.VMEM((DIM * TAIL,), jnp.float32),
        pltpu.SemaphoreType.DMA,
        pltpu.SemaphoreType.DMA,
        pltpu.SemaphoreType.DMA,
    ],
    compiler_params=pltpu.CompilerParams(use_tc_tiling_on_sc=True, needs_layout_passes=False),
)
def _emb_lookup(idx_hbm, tab_hbm, tails_hbm, out_hbm, idx_v, tab_v, out_v, tail_v,
                sem_t, sem_s0, sem_s1):
    wid = lax.axis_index("s") * NC + lax.axis_index("c")
    p0 = wid * PAIRS_PER_W
    sem_s = (sem_s0, sem_s1)

    def _store_drain(f, d, j):
        pltpu.make_async_copy(
            out_v.at[j % 2], out_hbm.at[f, d, pl.ds(j * QOUT, QOUT)], sem_s[j % 2]
        ).wait()

    def _pair(k, carry):
        p = p0 + k
        f = p // DIM
        d = lax.rem(p, DIM)

        tcps = [
            pltpu.async_copy(
                tab_hbm.at[f, d, pl.ds(off, sz)],
                tab_v.at[pl.ds(off, sz)],
                sem_t,
            )
            for off, sz in VCHS
        ]

        @pl.when(jnp.logical_or(k == 0, d == 0))
        def _load_idx():
            pltpu.sync_copy(idx_hbm.at[f], idx_v)
            pltpu.sync_copy(tails_hbm.at[f], tail_v)

        for t in range(TAIL // 16):
            tab_v[pl.ds(VTAIL + t * 16, 16)] = tail_v[pl.ds(d * TAIL + t * 16, 16)]

        for cp in tcps:
            cp.wait()

        for j in range(4):
            if j >= 2:
                _store_drain(f, d, j)
            else:
                @pl.when(k > 0)
                def _drain_prev():
                    _store_drain(f, d, j)
            base = j * QOUT

            @plsc.parallel_loop(0, G16, 1, unroll=16)
            def _g16(i):
                idx16 = idx_v[pl.ds(base + i * 16, 16)]
                out_v[j % 2, pl.ds(i * 16, 16)] = plsc.load_gather(tab_v, [idx16])
            pltpu.async_copy(
                out_v.at[j % 2], out_hbm.at[f, d, pl.ds(base, QOUT)], sem_s[j % 2]
            )
        return carry

    lax.fori_loop(0, PAIRS_PER_W, _pair, 0)
    pl_last = p0 + PAIRS_PER_W - 1
    _store_drain(pl_last // DIM, lax.rem(pl_last, DIM), 2)
    _store_drain(pl_last // DIM, lax.rem(pl_last, DIM), 3)


def kernel(sparse_inputs, tables):
    idx_t = sparse_inputs.astype(jnp.int32).T
    tab_t = jnp.transpose(tables, (0, 2, 1))
    tails = jnp.transpose(tables[:, VTAIL:, :], (0, 2, 1)).reshape(NUM_FIELDS,
                                                                   DIM * TAIL)
    out_t = _emb_lookup(idx_t, tab_t, tails)
    return jnp.transpose(out_t, (2, 0, 1))


# speedup vs baseline: 1.0029x; 1.0029x over previous
import functools

import jax
import jax.numpy as jnp
from jax import lax
from jax.experimental import pallas as pl
from jax.experimental.pallas import tpu as pltpu
from jax.experimental.pallas import tpu_sc as plsc

NUM_FIELDS = 26
VOCAB = 100000
DIM = 32
BATCH = 16384

NC = 2
NS = 16
NW = NC * NS
PAIRS_PER_W = (NUM_FIELDS * DIM) // NW
QOUT = BATCH // 4
G16 = QOUT // 16
NCH = 4
VCH = 12544
VTAIL = (VOCAB // 128) * 128
VCHS = [(i * VCH, VCH) for i in range(7)] + [(7 * VCH, VTAIL - 7 * VCH)]
TAIL = VOCAB - VTAIL

_mesh = plsc.VectorSubcoreMesh(core_axis_name="c", subcore_axis_name="s")


@functools.partial(
    pl.kernel,
    out_type=jax.ShapeDtypeStruct((NUM_FIELDS, DIM, BATCH), jnp.float32),
    mesh=_mesh,
    scratch_types=[
        pltpu.VMEM((BATCH,), jnp.int32),
        pltpu.VMEM((VOCAB,), jnp.float32),
        pltpu.VMEM((2, QOUT), jnp.float32),
        pltpu.VMEM((DIM * TAIL,), jnp.float32),
        pltpu.SemaphoreType.DMA,
        pltpu.SemaphoreType.DMA,
        pltpu.SemaphoreType.DMA,
    ],
    compiler_params=pltpu.CompilerParams(use_tc_tiling_on_sc=True, needs_layout_passes=False),
)
def _emb_lookup(idx_hbm, tab_hbm, tails_hbm, out_hbm, idx_v, tab_v, out_v, tail_v,
                sem_t, sem_s0, sem_s1):
    wid = lax.axis_index("s") * NC + lax.axis_index("c")
    p0 = wid * PAIRS_PER_W
    sem_s = (sem_s0, sem_s1)

    def _store_drain(f, d, j):
        pltpu.make_async_copy(
            out_v.at[j % 2], out_hbm.at[f, d, pl.ds(j * QOUT, QOUT)], sem_s[j % 2]
        ).wait()

    def _pair(k, carry):
        p = p0 + k
        f = p // DIM
        d = lax.rem(p, DIM)

        tcps = [
            pltpu.async_copy(
                tab_hbm.at[f, d, pl.ds(off, sz)],
                tab_v.at[pl.ds(off, sz)],
                sem_t,
            )
            for off, sz in VCHS
        ]

        @pl.when(jnp.logical_or(k == 0, d == 0))
        def _load_idx():
            pltpu.sync_copy(idx_hbm.at[f], idx_v)
            pltpu.sync_copy(tails_hbm.at[f], tail_v)

        for t in range(TAIL // 16):
            tab_v[pl.ds(VTAIL + t * 16, 16)] = tail_v[pl.ds(d * TAIL + t * 16, 16)]

        for cp in tcps:
            cp.wait()

        for j in range(4):
            if j >= 2:
                _store_drain(f, d, j)
            else:
                @pl.when(k > 0)
                def _drain_prev():
                    _store_drain(f, d, j)
            base = j * QOUT

            @plsc.parallel_loop(0, G16, 1, unroll=16)
            def _g16(i):
                idx16 = idx_v[pl.ds(base + i * 16, 16)]
                out_v[j % 2, pl.ds(i * 16, 16)] = plsc.load_gather(tab_v, [idx16])
            pltpu.async_copy(
                out_v.at[j % 2], out_hbm.at[f, d, pl.ds(base, QOUT)], sem_s[j % 2]
            )
        return carry

    lax.fori_loop(0, PAIRS_PER_W, _pair, 0)
    pl_last = p0 + PAIRS_PER_W - 1
    _store_drain(pl_last // DIM, lax.rem(pl_last, DIM), 2)
    _store_drain(pl_last // DIM, lax.rem(pl_last, DIM), 3)


def kernel(sparse_inputs, tables):
    idx_t = sparse_inputs.astype(jnp.int32).T
    tab_t = jnp.transpose(tables, (0, 2, 1))
    tails = jnp.transpose(tables[:, VTAIL:, :], (0, 2, 1)).reshape(NUM_FIELDS,
                                                                   DIM * TAIL)
    out_t = _emb_lookup(idx_t, tab_t, tails)
    return jnp.transpose(out_t, (2, 0, 1))
